# baseline (speedup 1.0000x reference)
import functools

import jax
import jax.numpy as jnp
from jax import lax
from jax.experimental import pallas as pl
from jax.experimental.pallas import tpu as pltpu
from jax.experimental.pallas import tpu_sc as plsc

VOCAB = 1000000
EMBED_DIM = 64
BATCH = 4096
HIST = 200

NUM_CORES = 2
NUM_SUBCORES = 16
NW = NUM_CORES * NUM_SUBCORES

TOT = BATCH * HIST
PER_W = TOT // NW
CHUNK = 128
NCH = PER_W // CHUNK
NBUF = 4


def _make_sc_gather():
    mesh = plsc.VectorSubcoreMesh(core_axis_name="c", subcore_axis_name="s")

    @functools.partial(
        pl.kernel,
        out_type=jax.ShapeDtypeStruct((TOT, EMBED_DIM), jnp.float32),
        mesh=mesh,
        compiler_params=pltpu.CompilerParams(use_tc_tiling_on_sc=False),
        scratch_types=[
            pltpu.VMEM((NCH, CHUNK), jnp.int32),
            *[pltpu.VMEM((CHUNK, EMBED_DIM), jnp.float32) for _ in range(NBUF)],
            *[pltpu.SemaphoreType.DMA for _ in range(NBUF)],
        ],
    )
    def sc_gather(idx_hbm, table_hbm, out_hbm, idx_v, *bufs_and_sems):
        bufs = bufs_and_sems[:NBUF]
        sems = bufs_and_sems[NBUF:]

        wid = lax.axis_index("s") * NUM_CORES + lax.axis_index("c")
        chunk0 = wid * NCH

        pltpu.sync_copy(idx_hbm.at[pl.ds(chunk0, NCH)], idx_v)

        for b in range(NBUF):
            pltpu.async_copy(table_hbm.at[idx_v.at[b]], bufs[b], sems[b])

        def body(g, _):
            for b in range(NBUF):
                j = g * NBUF + b
                pltpu.make_async_copy(
                    table_hbm.at[idx_v.at[j]], bufs[b], sems[b]
                ).wait()
                pltpu.sync_copy(
                    bufs[b], out_hbm.at[pl.ds((chunk0 + j) * CHUNK, CHUNK)]
                )

                @pl.when(j + NBUF < NCH)
                def _():
                    pltpu.async_copy(
                        table_hbm.at[idx_v.at[j + NBUF]], bufs[b], sems[b]
                    )

            return 0

        lax.fori_loop(0, NCH // NBUF, body, 0)

    return sc_gather


_sc_gather = _make_sc_gather()


@jax.jit
def kernel(input_indices, table):
    idx = input_indices.reshape(TOT // CHUNK, CHUNK).astype(jnp.int32)
    out = _sc_gather(idx, table)
    return out.reshape(BATCH, HIST, EMBED_DIM)


# speedup vs baseline: 1.2218x; 1.2218x over previous
import functools

import jax
import jax.numpy as jnp
from jax import lax
from jax.experimental import pallas as pl
from jax.experimental.pallas import tpu as pltpu
from jax.experimental.pallas import tpu_sc as plsc

VOCAB = 1000000
EMBED_DIM = 64
PADDED_DIM = 128
BATCH = 4096
HIST = 200

NUM_CORES = 2
NUM_SUBCORES = 16
NW = NUM_CORES * NUM_SUBCORES

TOT = BATCH * HIST
PER_W = TOT // NW
CHUNK = 128
NCH = PER_W // CHUNK
NBUF = 4


def _make_sc_gather():
    mesh = plsc.VectorSubcoreMesh(core_axis_name="c", subcore_axis_name="s")

    @functools.partial(
        pl.kernel,
        out_type=jax.ShapeDtypeStruct((TOT, PADDED_DIM), jnp.float32),
        mesh=mesh,
        scratch_types=[
            pltpu.VMEM((NCH, CHUNK), jnp.int32),
            *[pltpu.VMEM((CHUNK, PADDED_DIM), jnp.float32) for _ in range(NBUF)],
            *[pltpu.SemaphoreType.DMA for _ in range(NBUF)],
        ],
    )
    def sc_gather(idx_hbm, table_hbm, out_hbm, idx_v, *bufs_and_sems):
        bufs = bufs_and_sems[:NBUF]
        sems = bufs_and_sems[NBUF:]

        wid = lax.axis_index("s") * NUM_CORES + lax.axis_index("c")
        chunk0 = wid * NCH

        pltpu.sync_copy(idx_hbm.at[pl.ds(chunk0, NCH)], idx_v)

        for b in range(NBUF):
            pltpu.async_copy(table_hbm.at[idx_v.at[b]], bufs[b], sems[b])

        def body(g, _):
            for b in range(NBUF):
                j = g * NBUF + b
                pltpu.make_async_copy(
                    table_hbm.at[idx_v.at[j]], bufs[b], sems[b]
                ).wait()
                pltpu.sync_copy(
                    bufs[b],
                    out_hbm.at[pl.ds((chunk0 + j) * CHUNK, CHUNK)],
                )

                @pl.when(j + NBUF < NCH)
                def _():
                    pltpu.async_copy(
                        table_hbm.at[idx_v.at[j + NBUF]], bufs[b], sems[b]
                    )

            return 0

        lax.fori_loop(0, NCH // NBUF, body, 0)

    return sc_gather


_sc_gather = _make_sc_gather()


@jax.jit
def kernel(input_indices, table):
    table_padded = jnp.pad(table, ((0, 0), (0, PADDED_DIM - EMBED_DIM)))
    idx = input_indices.reshape(TOT // CHUNK, CHUNK)
    out = _sc_gather(idx, table_padded)
    return out[:, :EMBED_DIM].reshape(BATCH, HIST, EMBED_DIM)
